# Initial kernel scaffold; baseline (speedup 1.0000x reference)
import jax
import jax.numpy as jnp
from jax.experimental import pallas as pl


def kernel(x, W, costs):
    raise NotImplementedError("write your pallas kernel here")


# speedup vs baseline: 1.6947x; 1.6947x over previous
import jax
import jax.numpy as jnp
from jax.experimental import pallas as pl

BETA = 0.1
BLK = 512


def _router_body(x_ref, wt_ref, costs_ref, probs_ref, idx_ref, tpv_ref, cnt_ref):
    x = x_ref[...]
    wt = wt_ref[...]
    ne = wt.shape[1]
    logits = jnp.dot(x, wt, preferred_element_type=jnp.float32)
    logits = logits - BETA * costs_ref[...]
    m = jnp.max(logits, axis=-1, keepdims=True)
    e = jnp.exp(logits - m)
    s = jnp.sum(e, axis=-1, keepdims=True)
    probs = e / s
    probs_ref[...] = probs

    iota = jax.lax.broadcasted_iota(jnp.int32, probs.shape, 1)
    m1 = jnp.max(probs, axis=-1, keepdims=True)
    idx1 = jnp.min(jnp.where(probs == m1, iota, ne), axis=-1, keepdims=True)
    sel1 = iota == idx1
    masked = jnp.where(sel1, -1.0, probs)
    m2 = jnp.max(masked, axis=-1, keepdims=True)
    idx2 = jnp.min(jnp.where(masked == m2, iota, ne), axis=-1, keepdims=True)
    sel2 = iota == idx2

    denom = m1 + m2 + 1e-8
    idx_ref[:, 0:1] = idx1
    idx_ref[:, 1:2] = idx2
    tpv_ref[:, 0:1] = m1 / denom
    tpv_ref[:, 1:2] = m2 / denom

    cnt = jnp.sum(sel1.astype(jnp.int32) + sel2.astype(jnp.int32), axis=0,
                  keepdims=True)

    @pl.when(pl.program_id(0) == 0)
    def _init():
        cnt_ref[...] = cnt

    @pl.when(pl.program_id(0) != 0)
    def _acc():
        cnt_ref[...] += cnt


def kernel(x, W, costs):
    nt, embed = x.shape
    ne = W.shape[0]
    grid = (nt // BLK,)
    probs, idx, tpv, cnt = pl.pallas_call(
        _router_body,
        grid=grid,
        in_specs=[
            pl.BlockSpec((BLK, embed), lambda i: (i, 0)),
            pl.BlockSpec((embed, ne), lambda i: (0, 0)),
            pl.BlockSpec((1, ne), lambda i: (0, 0)),
        ],
        out_specs=[
            pl.BlockSpec((BLK, ne), lambda i: (i, 0)),
            pl.BlockSpec((BLK, 2), lambda i: (i, 0)),
            pl.BlockSpec((BLK, 2), lambda i: (i, 0)),
            pl.BlockSpec((1, ne), lambda i: (0, 0)),
        ],
        out_shape=[
            jax.ShapeDtypeStruct((nt, ne), jnp.float32),
            jax.ShapeDtypeStruct((nt, 2), jnp.int32),
            jax.ShapeDtypeStruct((nt, 2), jnp.float32),
            jax.ShapeDtypeStruct((1, ne), jnp.int32),
        ],
    )(x, W.T, costs.reshape(1, ne))
    return (idx, tpv, probs, cnt.reshape(-1))


# speedup vs baseline: 2.1135x; 1.2471x over previous
import jax
import jax.numpy as jnp
from jax.experimental import pallas as pl

BETA = 0.1
BLK = 1024


def _router_body(x_ref, wt_ref, costs_ref, probs_ref, idx_ref, tpv_ref, cnt_ref):
    x = x_ref[...]
    wt = wt_ref[...]
    ne = wt.shape[1]
    logits = jnp.dot(x, wt, preferred_element_type=jnp.float32)
    logits = logits - BETA * costs_ref[...]
    m = jnp.max(logits, axis=-1, keepdims=True)
    e = jnp.exp(logits - m)
    s = jnp.sum(e, axis=-1, keepdims=True)
    probs = e / s
    probs_ref[...] = probs

    iota = jax.lax.broadcasted_iota(jnp.int32, probs.shape, 1)
    m1 = jnp.max(probs, axis=-1, keepdims=True)
    idx1 = jnp.min(jnp.where(probs == m1, iota, ne), axis=-1, keepdims=True)
    sel1 = iota == idx1
    masked = jnp.where(sel1, -1.0, probs)
    m2 = jnp.max(masked, axis=-1, keepdims=True)
    idx2 = jnp.min(jnp.where(masked == m2, iota, ne), axis=-1, keepdims=True)
    sel2 = iota == idx2

    denom = m1 + m2 + 1e-8
    idx_ref[:, 0:1] = idx1
    idx_ref[:, 1:2] = idx2
    tpv_ref[:, 0:1] = m1 / denom
    tpv_ref[:, 1:2] = m2 / denom

    cnt = jnp.sum(sel1.astype(jnp.int32) + sel2.astype(jnp.int32), axis=0,
                  keepdims=True)

    @pl.when(pl.program_id(0) == 0)
    def _init():
        cnt_ref[...] = cnt

    @pl.when(pl.program_id(0) != 0)
    def _acc():
        cnt_ref[...] += cnt


def kernel(x, W, costs):
    nt, embed = x.shape
    ne = W.shape[0]
    grid = (nt // BLK,)
    probs, idx, tpv, cnt = pl.pallas_call(
        _router_body,
        grid=grid,
        in_specs=[
            pl.BlockSpec((BLK, embed), lambda i: (i, 0)),
            pl.BlockSpec((embed, ne), lambda i: (0, 0)),
            pl.BlockSpec((1, ne), lambda i: (0, 0)),
        ],
        out_specs=[
            pl.BlockSpec((BLK, ne), lambda i: (i, 0)),
            pl.BlockSpec((BLK, 2), lambda i: (i, 0)),
            pl.BlockSpec((BLK, 2), lambda i: (i, 0)),
            pl.BlockSpec((1, ne), lambda i: (0, 0)),
        ],
        out_shape=[
            jax.ShapeDtypeStruct((nt, ne), jnp.float32),
            jax.ShapeDtypeStruct((nt, 2), jnp.int32),
            jax.ShapeDtypeStruct((nt, 2), jnp.float32),
            jax.ShapeDtypeStruct((1, ne), jnp.int32),
        ],
    )(x, W.T, costs.reshape(1, ne))
    return (idx, tpv, probs, cnt.reshape(-1))


# speedup vs baseline: 2.3475x; 1.1107x over previous
import jax
import jax.numpy as jnp
from jax.experimental import pallas as pl

BETA = 0.1
BLK = 2048


def _router_body(x_ref, wt_ref, costs_ref, probs_ref, idx_ref, tpv_ref, cnt_ref):
    x = x_ref[...]
    wt = wt_ref[...]
    ne = wt.shape[1]
    logits = jnp.dot(x, wt, preferred_element_type=jnp.float32)
    logits = logits - BETA * costs_ref[...]
    m = jnp.max(logits, axis=-1, keepdims=True)
    e = jnp.exp(logits - m)
    s = jnp.sum(e, axis=-1, keepdims=True)
    probs = e / s
    probs_ref[...] = probs

    iota = jax.lax.broadcasted_iota(jnp.int32, probs.shape, 1)
    m1 = jnp.max(probs, axis=-1, keepdims=True)
    idx1 = jnp.min(jnp.where(probs == m1, iota, ne), axis=-1, keepdims=True)
    sel1 = iota == idx1
    masked = jnp.where(sel1, -1.0, probs)
    m2 = jnp.max(masked, axis=-1, keepdims=True)
    idx2 = jnp.min(jnp.where(masked == m2, iota, ne), axis=-1, keepdims=True)
    sel2 = iota == idx2

    denom = m1 + m2 + 1e-8
    idx_ref[:, 0:1] = idx1
    idx_ref[:, 1:2] = idx2
    tpv_ref[:, 0:1] = m1 / denom
    tpv_ref[:, 1:2] = m2 / denom

    cnt = jnp.sum(sel1.astype(jnp.int32) + sel2.astype(jnp.int32), axis=0,
                  keepdims=True)

    @pl.when(pl.program_id(0) == 0)
    def _init():
        cnt_ref[...] = cnt

    @pl.when(pl.program_id(0) != 0)
    def _acc():
        cnt_ref[...] += cnt


def kernel(x, W, costs):
    nt, embed = x.shape
    ne = W.shape[0]
    grid = (nt // BLK,)
    probs, idx, tpv, cnt = pl.pallas_call(
        _router_body,
        grid=grid,
        in_specs=[
            pl.BlockSpec((BLK, embed), lambda i: (i, 0)),
            pl.BlockSpec((embed, ne), lambda i: (0, 0)),
            pl.BlockSpec((1, ne), lambda i: (0, 0)),
        ],
        out_specs=[
            pl.BlockSpec((BLK, ne), lambda i: (i, 0)),
            pl.BlockSpec((BLK, 2), lambda i: (i, 0)),
            pl.BlockSpec((BLK, 2), lambda i: (i, 0)),
            pl.BlockSpec((1, ne), lambda i: (0, 0)),
        ],
        out_shape=[
            jax.ShapeDtypeStruct((nt, ne), jnp.float32),
            jax.ShapeDtypeStruct((nt, 2), jnp.int32),
            jax.ShapeDtypeStruct((nt, 2), jnp.float32),
            jax.ShapeDtypeStruct((1, ne), jnp.int32),
        ],
    )(x, W.T, costs.reshape(1, ne))
    return (idx, tpv, probs, cnt.reshape(-1))


# speedup vs baseline: 2.4576x; 1.0469x over previous
import jax
import jax.numpy as jnp
from jax.experimental import pallas as pl

BETA = 0.1
BLK = 4096


def _router_body(x_ref, wt_ref, costs_ref, probs_ref, idx_ref, tpv_ref, cnt_ref):
    x = x_ref[...]
    wt = wt_ref[...]
    ne = wt.shape[1]
    logits = jnp.dot(x, wt, preferred_element_type=jnp.float32)
    logits = logits - BETA * costs_ref[...]
    m = jnp.max(logits, axis=-1, keepdims=True)
    e = jnp.exp(logits - m)
    s = jnp.sum(e, axis=-1, keepdims=True)
    probs = e / s
    probs_ref[...] = probs

    iota = jax.lax.broadcasted_iota(jnp.int32, probs.shape, 1)
    m1 = jnp.max(probs, axis=-1, keepdims=True)
    idx1 = jnp.min(jnp.where(probs == m1, iota, ne), axis=-1, keepdims=True)
    sel1 = iota == idx1
    masked = jnp.where(sel1, -1.0, probs)
    m2 = jnp.max(masked, axis=-1, keepdims=True)
    idx2 = jnp.min(jnp.where(masked == m2, iota, ne), axis=-1, keepdims=True)
    sel2 = iota == idx2

    denom = m1 + m2 + 1e-8
    idx_ref[:, 0:1] = idx1
    idx_ref[:, 1:2] = idx2
    tpv_ref[:, 0:1] = m1 / denom
    tpv_ref[:, 1:2] = m2 / denom

    cnt = jnp.sum(sel1.astype(jnp.int32) + sel2.astype(jnp.int32), axis=0,
                  keepdims=True)

    @pl.when(pl.program_id(0) == 0)
    def _init():
        cnt_ref[...] = cnt

    @pl.when(pl.program_id(0) != 0)
    def _acc():
        cnt_ref[...] += cnt


def kernel(x, W, costs):
    nt, embed = x.shape
    ne = W.shape[0]
    grid = (nt // BLK,)
    probs, idx, tpv, cnt = pl.pallas_call(
        _router_body,
        grid=grid,
        in_specs=[
            pl.BlockSpec((BLK, embed), lambda i: (i, 0)),
            pl.BlockSpec((embed, ne), lambda i: (0, 0)),
            pl.BlockSpec((1, ne), lambda i: (0, 0)),
        ],
        out_specs=[
            pl.BlockSpec((BLK, ne), lambda i: (i, 0)),
            pl.BlockSpec((BLK, 2), lambda i: (i, 0)),
            pl.BlockSpec((BLK, 2), lambda i: (i, 0)),
            pl.BlockSpec((1, ne), lambda i: (0, 0)),
        ],
        out_shape=[
            jax.ShapeDtypeStruct((nt, ne), jnp.float32),
            jax.ShapeDtypeStruct((nt, 2), jnp.int32),
            jax.ShapeDtypeStruct((nt, 2), jnp.float32),
            jax.ShapeDtypeStruct((1, ne), jnp.int32),
        ],
    )(x, W.T, costs.reshape(1, ne))
    return (idx, tpv, probs, cnt.reshape(-1))
